# baseline (speedup 1.0000x reference)
import jax
import jax.numpy as jnp
from jax import lax
from jax.experimental import pallas as pl
from jax.experimental.pallas import tpu as pltpu
from jax.experimental.pallas import tpu_sc as plsc

_NC = 2
_NS = 16
_NW = _NC * _NS
_L = 16
_C = 8
_H = 768


def _rsqrt_sc(xv):
    iv = plsc.bitcast(xv, jnp.int32)
    y = plsc.bitcast(
        jnp.full((_L,), 0x5F3759DF, jnp.int32) - (iv >> 1), jnp.float32)
    hx = xv * 0.5
    y = y * (1.5 - hx * y * y)
    y = y * (1.5 - hx * y * y)
    y = y * (1.5 - hx * y * y)
    return y


def _sc_body(inp, pos, gam, bet, out,
             pos_v, sbuf, g_v, b_v,
             sem_in, sem_out, sem_pos):
    B, S, H = inp.shape
    wid = lax.axis_index("s") * _NC + lax.axis_index("c")
    s_per_w = S // _NW
    nchunk = s_per_w // _C
    base = wid * s_per_w
    pltpu.sync_copy(gam, g_v)
    pltpu.sync_copy(bet, b_v)
    lane = jnp.arange(_L, dtype=jnp.int32)
    perms = [lane ^ (1 << k) for k in range(4)]

    def in_dma(ci):
        p = lax.rem(ci, 2)
        return pltpu.make_async_copy(
            inp.at[:, pl.ds(base + ci * _C, _C)],
            sbuf.at[p], sem_in.at[p])

    def out_dma(ci):
        p = lax.rem(ci, 2)
        return pltpu.make_async_copy(
            sbuf.at[p],
            out.at[:, pl.ds(base + ci * _C, _C)], sem_out.at[p])

    def pos_dma(ci):
        p = lax.rem(ci, 2)
        return pltpu.make_async_copy(
            pos.at[pl.ds(base + ci * _C, _C)],
            pos_v.at[pl.ds(p * _C, _C)], sem_pos.at[p])

    pos_dma(0).start()
    pos_dma(1).start()
    in_dma(0).start()
    in_dma(1).start()

    def row_body2(p, i, _):
        poff = p * _C
        zero = jnp.zeros((_L,), jnp.float32)
        init = (zero,) * 8

        @plsc.parallel_loop(0, _H, _L, unroll=4, carry=init)
        def p1(j, accs):
            js = pl.ds(j, _L)
            pv = pos_v[poff + i, js]
            na = []
            ns = []
            for b in range(4):
                t = sbuf[p, b, i, js] + pv
                sbuf[p, b, i, js] = t
                na.append(accs[b] + t)
                ns.append(accs[4 + b] + t * t)
            return tuple(na) + tuple(ns)

        ys = []
        os_ = []
        for b in range(4):
            acc = p1[b]
            sq = p1[4 + b]
            for p16 in perms:
                acc = acc + jnp.take_along_axis(acc, p16, axis=0)
                sq = sq + jnp.take_along_axis(sq, p16, axis=0)
            m = acc * (1.0 / _H)
            y = _rsqrt_sc(sq * (1.0 / _H) - m * m + 1e-5)
            ys.append(y)
            os_.append(m * y)

        @plsc.parallel_loop(0, _H, _L, unroll=4)
        def p2(j):
            js = pl.ds(j, _L)
            g = g_v[js]
            bb = b_v[js]
            for b in range(4):
                t = sbuf[p, b, i, js]
                sbuf[p, b, i, js] = (t * ys[b] - os_[b]) * g + bb

        return 0

    def chunk_body2(ci, _):
        p = lax.rem(ci, 2)
        pos_dma(ci).wait()
        in_dma(ci).wait()

        lax.fori_loop(0, 0, lambda i, c: row_body2(p, i, c), 0)

        @pl.when(jnp.logical_and(ci >= 1, ci + 1 < nchunk))
        def _():
            out_dma(ci - 1).wait()
            in_dma(ci + 1).start()

        lax.fori_loop(0, 0, lambda i, c: row_body2(p, i, c), 0)

        out_dma(ci).start()

        @pl.when(ci + 2 < nchunk)
        def _():
            pos_dma(ci + 2).start()

        return 0

    lax.fori_loop(0, nchunk, chunk_body2, 0)
    out_dma(nchunk - 2).wait()
    out_dma(nchunk - 1).wait()


def _sc_layernorm(input_feat, pos_table, ln_gamma, ln_beta):
    B, S, H = input_feat.shape
    mesh = plsc.VectorSubcoreMesh(core_axis_name="c", subcore_axis_name="s")
    fn = pl.kernel(
        _sc_body,
        mesh=mesh,
        compiler_params=pltpu.CompilerParams(
            use_tc_tiling_on_sc=False, needs_layout_passes=False),
        out_type=jax.ShapeDtypeStruct((B, S, H), jnp.float32),
        scratch_types=[
            pltpu.VMEM((2 * _C, H), jnp.float32),
            pltpu.VMEM((2, B, _C, H), jnp.float32),
            pltpu.VMEM((H,), jnp.float32),
            pltpu.VMEM((H,), jnp.float32),
            pltpu.SemaphoreType.DMA((2,)),
            pltpu.SemaphoreType.DMA((2,)),
            pltpu.SemaphoreType.DMA((2,)),
        ],
    )
    return fn(input_feat, pos_table, ln_gamma, ln_beta)


def kernel(input_feat, pos_table, ln_gamma, ln_beta):
    return _sc_layernorm(input_feat, pos_table, ln_gamma, ln_beta)


# speedup vs baseline: 4.1904x; 4.1904x over previous
import jax
import jax.numpy as jnp
from jax import lax
from jax.experimental import pallas as pl
from jax.experimental.pallas import tpu as pltpu

_R = 256
_EPS = 1e-5


def _tc_body(x_ref, pos_ref, g_ref, b_ref, o_ref):
    x = x_ref[...] + pos_ref[...][None, :, :]
    m = jnp.mean(x, axis=-1, keepdims=True)
    xc = x - m
    var = jnp.mean(xc * xc, axis=-1, keepdims=True)
    o_ref[...] = xc * lax.rsqrt(var + _EPS) * g_ref[...] + b_ref[...]


def _tc_layernorm(input_feat, pos_slice, ln_gamma, ln_beta):
    B, S, H = input_feat.shape
    grid = (S // _R,)
    return pl.pallas_call(
        _tc_body,
        grid=grid,
        in_specs=[
            pl.BlockSpec((B, _R, H), lambda i: (0, i, 0)),
            pl.BlockSpec((_R, H), lambda i: (i, 0)),
            pl.BlockSpec((H,), lambda i: (0,)),
            pl.BlockSpec((H,), lambda i: (0,)),
        ],
        out_specs=pl.BlockSpec((B, _R, H), lambda i: (0, i, 0)),
        out_shape=jax.ShapeDtypeStruct((B, S, H), jnp.float32),
    )(input_feat, pos_slice, ln_gamma, ln_beta)


def kernel(input_feat, pos_table, ln_gamma, ln_beta):
    B, S, H = input_feat.shape
    pos_slice = lax.slice(pos_table, (0, 0), (S, H))
    return _tc_layernorm(input_feat, pos_slice, ln_gamma, ln_beta)


# speedup vs baseline: 4.4062x; 1.0515x over previous
import jax
import jax.numpy as jnp
from jax import lax
from jax.experimental import pallas as pl
from jax.experimental.pallas import tpu as pltpu

_R = 512
_EPS = 1e-5


def _tc_body(x_ref, pos_ref, g_ref, b_ref, o_ref):
    x = x_ref[...] + pos_ref[...][None, :, :]
    m = jnp.mean(x, axis=-1, keepdims=True)
    xc = x - m
    var = jnp.mean(xc * xc, axis=-1, keepdims=True)
    o_ref[...] = xc * lax.rsqrt(var + _EPS) * g_ref[...] + b_ref[...]


def _tc_layernorm(input_feat, pos_slice, ln_gamma, ln_beta):
    B, S, H = input_feat.shape
    grid = (S // _R,)
    return pl.pallas_call(
        _tc_body,
        grid=grid,
        in_specs=[
            pl.BlockSpec((B, _R, H), lambda i: (0, i, 0)),
            pl.BlockSpec((_R, H), lambda i: (i, 0)),
            pl.BlockSpec((H,), lambda i: (0,)),
            pl.BlockSpec((H,), lambda i: (0,)),
        ],
        out_specs=pl.BlockSpec((B, _R, H), lambda i: (0, i, 0)),
        out_shape=jax.ShapeDtypeStruct((B, S, H), jnp.float32),
    )(input_feat, pos_slice, ln_gamma, ln_beta)


def kernel(input_feat, pos_table, ln_gamma, ln_beta):
    B, S, H = input_feat.shape
    pos_slice = lax.slice(pos_table, (0, 0), (S, H))
    return _tc_layernorm(input_feat, pos_slice, ln_gamma, ln_beta)
